# Initial kernel scaffold; baseline (speedup 1.0000x reference)
import jax
import jax.numpy as jnp
from jax.experimental import pallas as pl


def kernel(scores, decoded_mask, token_ids, beam_scores):
    raise NotImplementedError("write your pallas kernel here")


# speedup vs baseline: 1.2016x; 1.2016x over previous
import functools

import jax
import jax.numpy as jnp
from jax import lax
from jax.experimental import pallas as pl
from jax.experimental.pallas import tpu as pltpu
from jax.experimental.pallas import tpu_sc as plsc

_B = 128
_V = 100000
_HIST = 32
_PEN = 1.2
_ROWS = 8
_GRID = _B // _ROWS
_K = 5
_LANES = 16


def _sc_penalize(scores, token_ids):
    info = plsc.get_sparse_core_info()
    num_workers = info.num_cores * info.num_subcores
    rows_per_w = _B // num_workers
    mesh = plsc.VectorSubcoreMesh(core_axis_name="c", subcore_axis_name="s")

    @functools.partial(
        pl.kernel,
        mesh=mesh,
        out_type=jax.ShapeDtypeStruct((_B, _V), jnp.float32),
        scratch_types=[
            pltpu.VMEM((_V,), jnp.float32),
            pltpu.VMEM((_HIST,), jnp.int32),
        ],
        compiler_params=pltpu.CompilerParams(needs_layout_passes=False),
    )
    def sc_kernel(scores_hbm, tid_hbm, out_hbm, row_v, tid_v):
        wid = lax.axis_index("s") * info.num_cores + lax.axis_index("c")
        for r in range(rows_per_w):
            b = wid * rows_per_w + r
            pltpu.sync_copy(scores_hbm.at[b], row_v)
            pltpu.sync_copy(tid_hbm.at[b], tid_v)
            gathered = []
            for c in range(_HIST // _LANES):
                idx = tid_v[pl.ds(c * _LANES, _LANES)]
                gathered.append((idx, plsc.load_gather(row_v, [idx])))
            for idx, v in gathered:
                pv = jnp.where(v < 0.0, v * _PEN, v * (1.0 / _PEN))
                plsc.store_scatter(row_v, [idx], pv)
            pltpu.sync_copy(row_v, out_hbm.at[b])

    return sc_kernel(scores, token_ids)


_NEG = -3e38
_MASKED_OUT = -1e24


def _tc_body(x_ref, mk_ref, beam_ref, outv_ref, outt_ref):
    x = x_ref[...]
    mk = mk_ref[...] != 0
    m = jnp.max(x, axis=1, keepdims=True)
    s = jnp.sum(jnp.exp(x - m), axis=1, keepdims=True)
    lse = m + jnp.log(s)
    beam = jnp.reshape(beam_ref[...], (_ROWS, 1))
    key = jnp.where(mk, x, jnp.float32(_NEG))
    iota = lax.broadcasted_iota(jnp.int32, (_ROWS, _V), 1)
    big = jnp.int32(2**31 - 1)
    vals, toks = [], []
    for k in range(_K):
        mx = jnp.max(key, axis=1, keepdims=True)
        eq = key == mx
        pos = jnp.min(jnp.where(eq, iota, big), axis=1, keepdims=True)
        val = jnp.where(mx < jnp.float32(-1e29), jnp.float32(_MASKED_OUT), mx - lse) + beam
        vals.append(val)
        toks.append(pos)
        if k < _K - 1:
            key = jnp.where(iota == pos, -jnp.inf, key)
    outv_ref[...] = jnp.concatenate(vals, axis=1)
    outt_ref[...] = jnp.concatenate(toks, axis=1)


def _tc_topk(xmod, mask_i32, beam3d):
    return pl.pallas_call(
        _tc_body,
        grid=(_GRID,),
        in_specs=[
            pl.BlockSpec((_ROWS, _V), lambda i: (i, 0)),
            pl.BlockSpec((_ROWS, _V), lambda i: (i, 0)),
            pl.BlockSpec((1, 1, _ROWS), lambda i: (i, 0, 0)),
        ],
        out_specs=[
            pl.BlockSpec((_ROWS, _K), lambda i: (i, 0)),
            pl.BlockSpec((_ROWS, _K), lambda i: (i, 0)),
        ],
        out_shape=[
            jax.ShapeDtypeStruct((_B, _K), jnp.float32),
            jax.ShapeDtypeStruct((_B, _K), jnp.int32),
        ],
    )(xmod, mask_i32, beam3d)


def kernel(scores, decoded_mask, token_ids, beam_scores):
    scores = scores.astype(jnp.float32)
    tid = token_ids.astype(jnp.int32)
    mk = decoded_mask.astype(jnp.int32)
    beam = beam_scores.astype(jnp.float32).reshape(_GRID, 1, _ROWS)
    xmod = _sc_penalize(scores, tid)
    next_scores, next_tokens = _tc_topk(xmod, mk, beam)
    return next_scores, next_tokens
